# baseline (speedup 1.0000x reference)
import functools

import jax
import jax.numpy as jnp
from jax import lax
from jax.experimental import pallas as pl
from jax.experimental.pallas import tpu as pltpu
from jax.experimental.pallas import tpu_sc as plsc

_V = 1000000
_D = 64
_DP = 128
_ROWS, _SEQ = 4096, 200
_B = _ROWS * _SEQ
_NC, _NS = 2, 16
_NW = _NC * _NS

_W = 128
_NFULL = _V // _W
_VTAIL = _NFULL * _W

_CHUNK = 128
_NCH = _B // _CHUNK // _NW
_BBLK = _ROWS // _CHUNK

_MESH = plsc.VectorSubcoreMesh(
    core_axis_name="c", subcore_axis_name="s", num_cores=_NC, num_subcores=_NS
)
_PARAMS = pltpu.CompilerParams(
    use_tc_tiling_on_sc=True, needs_layout_passes=False
)


def _fmt_body(tableT_hbm, tail_hbm, fmt_hbm, bufs, tbufs, rsems, wsems):
    wid = lax.axis_index("s") * _NC + lax.axis_index("c")
    lanes = lax.iota(jnp.int32, 16)
    vvecs = [lanes + 16 * m for m in range(_W // 16)]
    zeros = lanes * 0

    def transpose(src, dst):
        def step(d, carry):
            dvec = zeros + d
            for m in range(_W // 16):
                vals = src[d, pl.ds(16 * m, 16)]
                plsc.store_scatter(dst, [vvecs[m], dvec], vals)
            return carry

        lax.fori_loop(0, 64, step, 0)

    njobs = (_NFULL + _NW - 1) // _NW

    def read(job, p):
        strip = job * _NW + wid

        @pl.when(strip < _NFULL)
        def _():
            pltpu.make_async_copy(
                tableT_hbm.at[pl.ds(0, 64), pl.ds(strip * _W, _W)],
                bufs[p],
                rsems[p],
            ).start()

    def retire(job, p):
        strip = job * _NW + wid

        @pl.when(strip < _NFULL)
        def _():
            pltpu.make_async_copy(
                tableT_hbm.at[pl.ds(0, 64), pl.ds(strip * _W, _W)],
                bufs[p],
                rsems[p],
            ).wait()

            @pl.when(job >= 2)
            def _():
                pltpu.make_async_copy(
                    tbufs[p].at[:, pl.ds(0, _DP)],
                    fmt_hbm.at[pl.ds(0, _W)],
                    wsems[p],
                ).wait()

            transpose(bufs[p], tbufs[p])
            pltpu.make_async_copy(
                tbufs[p].at[:, pl.ds(0, _DP)],
                fmt_hbm.at[pl.ds(strip * _W, _W)],
                wsems[p],
            ).start()

    read(0, 0)
    read(1, 1)

    def job_loop(j, carry):
        for p in range(2):
            job = j * 2 + p
            retire(job, p)

            @pl.when(job + 2 < njobs)
            def _():
                read(job + 2, p)

        return carry

    lax.fori_loop(0, (njobs + 1) // 2, job_loop, 0)

    for p in range(2):
        pltpu.make_async_copy(
            tbufs[p].at[:, pl.ds(0, _DP)], fmt_hbm.at[pl.ds(0, _W)], wsems[p]
        ).wait()

    @pl.when(wid == 0)
    def _():
        pltpu.sync_copy(tail_hbm, bufs[0])
        pltpu.sync_copy(bufs[0], fmt_hbm.at[pl.ds(_VTAIL, 64)])


@functools.partial(
    pl.kernel,
    out_type=jax.ShapeDtypeStruct((_V, _DP), jnp.float32),
    mesh=_MESH,
    scratch_types=(
        [pltpu.VMEM((64, _W), jnp.float32) for _ in range(2)]
        + [pltpu.VMEM((_W, _DP + 1), jnp.float32) for _ in range(2)]
        + [pltpu.SemaphoreType.DMA for _ in range(4)]
    ),
    compiler_params=_PARAMS,
)
def _format_table(tableT_hbm, tail_hbm, fmt_hbm, b0, b1, t0, t1, r0, r1, w0, w1):
    _fmt_body(tableT_hbm, tail_hbm, fmt_hbm, (b0, b1), (t0, t1), (r0, r1), (w0, w1))


def _gather_body(idx_hbm, fmt_hbm, out_hbm, idx_v, *bufs_sems):
    gbufs = bufs_sems[0:2]
    tbufs = bufs_sems[2:4]
    gsems = bufs_sems[4:6]
    wsems = bufs_sems[6:8]
    wid = lax.axis_index("s") * _NC + lax.axis_index("c")
    base = wid * _NCH * _CHUNK
    lanes = lax.iota(jnp.int32, 16)
    dvecs = [lanes + 16 * m for m in range(_D // 16)]
    zeros = lanes * 0

    pltpu.sync_copy(idx_hbm.at[pl.ds(base, _NCH * _CHUNK)], idx_v)

    def transpose(src, dst):
        def step(l, carry):
            lvec = zeros + l
            for m in range(_D // 16):
                vals = src[l, pl.ds(16 * m, 16)]
                plsc.store_scatter(dst, [dvecs[m], lvec], vals)
            return carry

        lax.fori_loop(0, _CHUNK, step, 0)

    def gather(c, p):
        return pltpu.make_async_copy(
            fmt_hbm.at[idx_v.at[pl.ds(c * _CHUNK, _CHUNK)]], gbufs[p], gsems[p]
        )

    def write(c, p):
        chunk = wid * _NCH + c
        s = chunk // _BBLK
        bb = chunk % _BBLK
        return pltpu.make_async_copy(
            tbufs[p].at[:, pl.ds(0, _CHUNK)],
            out_hbm.at[s, pl.ds(0, _D), pl.ds(bb * _CHUNK, _CHUNK)],
            wsems[p],
        )

    def visit(c, p):
        @pl.when(c < _NCH)
        def _():
            gather(c, p).start()

        cp = c - 1
        q = 1 - p

        @pl.when(jnp.logical_and(cp >= 0, cp < _NCH))
        def _():
            gather(cp, q).wait()

            @pl.when(cp >= 2)
            def _():
                write(cp - 2, q).wait()

            transpose(gbufs[q], tbufs[q])
            write(cp, q).start()

    def outer(j, carry):
        for p in range(2):
            visit(j * 2 + p, p)
        return carry

    lax.fori_loop(0, (_NCH + 2) // 2 + 1, outer, 0)

    for p in range(2):
        write(_NCH - 2 + p, p).wait()


@functools.partial(
    pl.kernel,
    out_type=jax.ShapeDtypeStruct((_SEQ, _D, _ROWS), jnp.float32),
    mesh=_MESH,
    scratch_types=(
        [pltpu.VMEM((_NCH * _CHUNK,), jnp.int32)]
        + [pltpu.VMEM((_CHUNK, _DP), jnp.float32) for _ in range(2)]
        + [pltpu.VMEM((_D, _CHUNK + 1), jnp.float32) for _ in range(2)]
        + [pltpu.SemaphoreType.DMA for _ in range(4)]
    ),
    compiler_params=_PARAMS,
)
def _gather_rows2(idx_hbm, fmt_hbm, out_hbm, idx_v, *bufs_sems):
    _gather_body(idx_hbm, fmt_hbm, out_hbm, idx_v, *bufs_sems)


def kernel(input, table):
    tailp = jnp.pad(table[_VTAIL:], ((0, 0), (0, _DP - _D)))
    fmt = _format_table(table.T, tailp)
    idxT = input.T.reshape(-1).astype(jnp.int32)
    outT = _gather_rows2(idxT, fmt)
    return outT.transpose(2, 0, 1)


# speedup vs baseline: 1.0115x; 1.0115x over previous
import functools

import jax
import jax.numpy as jnp
from jax import lax
from jax.experimental import pallas as pl
from jax.experimental.pallas import tpu as pltpu
from jax.experimental.pallas import tpu_sc as plsc

_V = 1000000
_D = 64
_DP = 128
_ROWS, _SEQ = 4096, 200
_B = _ROWS * _SEQ
_NC, _NS = 2, 16
_NW = _NC * _NS

_W = 128
_NFULL = _V // _W
_VTAIL = _NFULL * _W

_CHUNK = 128
_NCH = _B // _CHUNK // _NW
_BBLK = _ROWS // _CHUNK

_MESH = plsc.VectorSubcoreMesh(
    core_axis_name="c", subcore_axis_name="s", num_cores=_NC, num_subcores=_NS
)
_PARAMS = pltpu.CompilerParams(
    use_tc_tiling_on_sc=True, needs_layout_passes=False
)


def _fmt_body(tableT_hbm, tail_hbm, fmt_hbm, bufs, tbufs, rsems, wsems):
    wid = lax.axis_index("s") * _NC + lax.axis_index("c")
    lanes = lax.iota(jnp.int32, 16)
    vvecs = [lanes + 16 * m for m in range(_W // 16)]
    zeros = lanes * 0

    def transpose(src, dst):
        def step(d, carry):
            dvec = zeros + d
            vals = [src[d, pl.ds(16 * m, 16)] for m in range(_W // 16)]
            for m in range(_W // 16):
                plsc.store_scatter(dst, [vvecs[m], dvec], vals[m])
            return carry

        lax.fori_loop(0, 64, step, 0)

    njobs = (_NFULL + _NW - 1) // _NW

    def read(job, p):
        strip = job * _NW + wid

        @pl.when(strip < _NFULL)
        def _():
            pltpu.make_async_copy(
                tableT_hbm.at[pl.ds(0, 64), pl.ds(strip * _W, _W)],
                bufs[p],
                rsems[p],
            ).start()

    def retire(job, p):
        strip = job * _NW + wid

        @pl.when(strip < _NFULL)
        def _():
            pltpu.make_async_copy(
                tableT_hbm.at[pl.ds(0, 64), pl.ds(strip * _W, _W)],
                bufs[p],
                rsems[p],
            ).wait()

            @pl.when(job >= 2)
            def _():
                pltpu.make_async_copy(
                    tbufs[p].at[:, pl.ds(0, _DP)],
                    fmt_hbm.at[pl.ds(0, _W)],
                    wsems[p],
                ).wait()

            transpose(bufs[p], tbufs[p])
            pltpu.make_async_copy(
                tbufs[p].at[:, pl.ds(0, _DP)],
                fmt_hbm.at[pl.ds(strip * _W, _W)],
                wsems[p],
            ).start()

    read(0, 0)
    read(1, 1)

    def job_loop(j, carry):
        for p in range(2):
            job = j * 2 + p
            retire(job, p)

            @pl.when(job + 2 < njobs)
            def _():
                read(job + 2, p)

        return carry

    lax.fori_loop(0, (njobs + 1) // 2, job_loop, 0)

    for p in range(2):
        pltpu.make_async_copy(
            tbufs[p].at[:, pl.ds(0, _DP)], fmt_hbm.at[pl.ds(0, _W)], wsems[p]
        ).wait()

    @pl.when(wid == 0)
    def _():
        pltpu.sync_copy(tail_hbm, bufs[0])
        pltpu.sync_copy(bufs[0], fmt_hbm.at[pl.ds(_VTAIL, 64)])


@functools.partial(
    pl.kernel,
    out_type=jax.ShapeDtypeStruct((_V, _DP), jnp.float32),
    mesh=_MESH,
    scratch_types=(
        [pltpu.VMEM((64, _W), jnp.float32) for _ in range(2)]
        + [pltpu.VMEM((_W, _DP + 1), jnp.float32) for _ in range(2)]
        + [pltpu.SemaphoreType.DMA for _ in range(4)]
    ),
    compiler_params=_PARAMS,
)
def _format_table(tableT_hbm, tail_hbm, fmt_hbm, b0, b1, t0, t1, r0, r1, w0, w1):
    _fmt_body(tableT_hbm, tail_hbm, fmt_hbm, (b0, b1), (t0, t1), (r0, r1), (w0, w1))


def _gather_body(idx_hbm, fmt_hbm, out_hbm, idx_v, *bufs_sems):
    gbufs = bufs_sems[0:2]
    tbufs = bufs_sems[2:4]
    gsems = bufs_sems[4:6]
    wsems = bufs_sems[6:8]
    wid = lax.axis_index("s") * _NC + lax.axis_index("c")
    base = wid * _NCH * _CHUNK
    lanes = lax.iota(jnp.int32, 16)
    dvecs = [lanes + 16 * m for m in range(_D // 16)]
    zeros = lanes * 0

    pltpu.sync_copy(idx_hbm.at[pl.ds(base, _NCH * _CHUNK)], idx_v)

    def transpose(src, dst):
        def step(l2, carry):
            l0 = 2 * l2
            lvecs_d = [zeros + (l0 + li) for li in range(2)]
            vals = [
                [src[l0 + li, pl.ds(16 * m, 16)] for m in range(_D // 16)]
                for li in range(2)
            ]
            for li in range(2):
                for m in range(_D // 16):
                    plsc.store_scatter(dst, [dvecs[m], lvecs_d[li]], vals[li][m])
            return carry

        lax.fori_loop(0, _CHUNK // 2, step, 0)

    def gather(c, p):
        return pltpu.make_async_copy(
            fmt_hbm.at[idx_v.at[pl.ds(c * _CHUNK, _CHUNK)]], gbufs[p], gsems[p]
        )

    def write(c, p):
        chunk = wid * _NCH + c
        s = chunk // _BBLK
        bb = chunk % _BBLK
        return pltpu.make_async_copy(
            tbufs[p].at[:, pl.ds(0, _CHUNK)],
            out_hbm.at[s, pl.ds(0, _D), pl.ds(bb * _CHUNK, _CHUNK)],
            wsems[p],
        )

    def visit(c, p):
        @pl.when(c < _NCH)
        def _():
            gather(c, p).start()

        cp = c - 1
        q = 1 - p

        @pl.when(jnp.logical_and(cp >= 0, cp < _NCH))
        def _():
            gather(cp, q).wait()

            @pl.when(cp >= 2)
            def _():
                write(cp - 2, q).wait()

            transpose(gbufs[q], tbufs[q])
            write(cp, q).start()

    def outer(j, carry):
        for p in range(2):
            visit(j * 2 + p, p)
        return carry

    lax.fori_loop(0, (_NCH + 2) // 2 + 1, outer, 0)

    for p in range(2):
        write(_NCH - 2 + p, p).wait()


@functools.partial(
    pl.kernel,
    out_type=jax.ShapeDtypeStruct((_SEQ, _D, _ROWS), jnp.float32),
    mesh=_MESH,
    scratch_types=(
        [pltpu.VMEM((_NCH * _CHUNK,), jnp.int32)]
        + [pltpu.VMEM((_CHUNK, _DP), jnp.float32) for _ in range(2)]
        + [pltpu.VMEM((_D, _CHUNK + 1), jnp.float32) for _ in range(2)]
        + [pltpu.SemaphoreType.DMA for _ in range(4)]
    ),
    compiler_params=_PARAMS,
)
def _gather_rows2(idx_hbm, fmt_hbm, out_hbm, idx_v, *bufs_sems):
    _gather_body(idx_hbm, fmt_hbm, out_hbm, idx_v, *bufs_sems)


def kernel(input, table):
    tailp = jnp.pad(table[_VTAIL:], ((0, 0), (0, _DP - _D)))
    fmt = _format_table(table.T, tailp)
    idxT = input.T.reshape(-1).astype(jnp.int32)
    outT = _gather_rows2(idxT, fmt)
    return outT.transpose(2, 0, 1)


# speedup vs baseline: 1.8001x; 1.7796x over previous
import functools

import jax
import jax.numpy as jnp
from jax import lax
from jax.experimental import pallas as pl
from jax.experimental.pallas import tpu as pltpu
from jax.experimental.pallas import tpu_sc as plsc

_D = 64
_B = 4096 * 200
_NC, _NS = 2, 16
_NW = _NC * _NS
_BW = _B // _NW
_ROWS = 4096
_SEQ = 200
_CHUNK = _SEQ
_CHUNKS = _BW // _CHUNK
_NSLOT = 6
_LAG = 3


def _body(idx_hbm, table_hbm, out_hbm, idx_v, *bufs_sems):
    bufs = bufs_sems[:_NSLOT]
    gsems = bufs_sems[_NSLOT : 2 * _NSLOT]
    wsems = bufs_sems[2 * _NSLOT :]
    wid = lax.axis_index("s") * _NC + lax.axis_index("c")
    base = wid * _CHUNKS

    pltpu.sync_copy(idx_hbm.at[wid], idx_v)

    def gather(c, b):
        return pltpu.make_async_copy(table_hbm.at[idx_v.at[c]], bufs[b], gsems[b])

    def write(c, b):
        return pltpu.make_async_copy(bufs[b], out_hbm.at[base + c], wsems[b])

    def visit(v, b):
        @pl.when(jnp.logical_and(v >= _NSLOT, v - _NSLOT < _CHUNKS))
        def _():
            write(v - _NSLOT, b).wait()

        @pl.when(v < _CHUNKS)
        def _():
            gather(v, b).start()

        b2 = (b - _LAG) % _NSLOT

        @pl.when(jnp.logical_and(v >= _LAG, v - _LAG < _CHUNKS))
        def _():
            gather(v - _LAG, b2).wait()
            write(v - _LAG, b2).start()

    def outer(g, carry):
        v0 = g * _NSLOT
        for j in range(_NSLOT):
            visit(v0 + j, j)
        return carry

    lax.fori_loop(0, (_CHUNKS + _NSLOT) // _NSLOT, outer, 0)


@functools.partial(
    pl.kernel,
    out_type=jax.ShapeDtypeStruct((_ROWS, _SEQ, _D), jnp.float32),
    mesh=plsc.VectorSubcoreMesh(
        core_axis_name="c", subcore_axis_name="s", num_cores=_NC, num_subcores=_NS
    ),
    scratch_types=(
        [pltpu.VMEM((_CHUNKS, _CHUNK), jnp.int32)]
        + [pltpu.VMEM((_CHUNK, _D), jnp.float32) for _ in range(_NSLOT)]
        + [pltpu.SemaphoreType.DMA for _ in range(2 * _NSLOT)]
    ),
    compiler_params=pltpu.CompilerParams(use_tc_tiling_on_sc=False),
)
def _gather_rows(idx_hbm, table_hbm, out_hbm, idx_v, *bufs_sems):
    _body(idx_hbm, table_hbm, out_hbm, idx_v, *bufs_sems)


def kernel(input, table):
    idx = input.reshape(_NW, _CHUNKS, _CHUNK).astype(jnp.int32)
    return _gather_rows(idx, table)


# speedup vs baseline: 1.8495x; 1.0275x over previous
import functools

import jax
import jax.numpy as jnp
from jax import lax
from jax.experimental import pallas as pl
from jax.experimental.pallas import tpu as pltpu
from jax.experimental.pallas import tpu_sc as plsc

_D = 64
_B = 4096 * 200
_NC, _NS = 2, 16
_NW = _NC * _NS
_BW = _B // _NW
_ROWS = 4096
_SEQ = 200
_CHUNK = 256
_BBLK = _ROWS // _CHUNK
_CHUNKS = _BW // _CHUNK
_NSLOT = 6
_LAG = 3


def _body(idx_hbm, table_hbm, out_hbm, idx_v, *bufs_sems):
    bufs = bufs_sems[:_NSLOT]
    gsems = bufs_sems[_NSLOT : 2 * _NSLOT]
    wsems = bufs_sems[2 * _NSLOT :]
    wid = lax.axis_index("s") * _NC + lax.axis_index("c")
    base = wid * _CHUNKS

    pltpu.sync_copy(idx_hbm.at[wid], idx_v)

    def gather(c, b):
        return pltpu.make_async_copy(table_hbm.at[idx_v.at[c]], bufs[b], gsems[b])

    def write(c, b):
        g = base + c
        s = g // _BBLK
        bb = g % _BBLK
        return pltpu.make_async_copy(
            bufs[b], out_hbm.at[s, pl.ds(bb * _CHUNK, _CHUNK)], wsems[b]
        )

    def visit(v, b):
        @pl.when(jnp.logical_and(v >= _NSLOT, v - _NSLOT < _CHUNKS))
        def _():
            write(v - _NSLOT, b).wait()

        @pl.when(v < _CHUNKS)
        def _():
            gather(v, b).start()

        b2 = (b - _LAG) % _NSLOT

        @pl.when(jnp.logical_and(v >= _LAG, v - _LAG < _CHUNKS))
        def _():
            gather(v - _LAG, b2).wait()
            write(v - _LAG, b2).start()

    def outer(g, carry):
        v0 = g * _NSLOT
        for j in range(_NSLOT):
            visit(v0 + j, j)
        return carry

    lax.fori_loop(0, (_CHUNKS + _NSLOT) // _NSLOT + 1, outer, 0)


@functools.partial(
    pl.kernel,
    out_type=jax.ShapeDtypeStruct((_SEQ, _ROWS, _D), jnp.float32),
    mesh=plsc.VectorSubcoreMesh(
        core_axis_name="c", subcore_axis_name="s", num_cores=_NC, num_subcores=_NS
    ),
    scratch_types=(
        [pltpu.VMEM((_CHUNKS, _CHUNK), jnp.int32)]
        + [pltpu.VMEM((_CHUNK, _D), jnp.float32) for _ in range(_NSLOT)]
        + [pltpu.SemaphoreType.DMA for _ in range(2 * _NSLOT)]
    ),
    compiler_params=pltpu.CompilerParams(use_tc_tiling_on_sc=False),
)
def _gather_rows(idx_hbm, table_hbm, out_hbm, idx_v, *bufs_sems):
    _body(idx_hbm, table_hbm, out_hbm, idx_v, *bufs_sems)


def kernel(input, table):
    idx = input.T.reshape(_NW, _CHUNKS, _CHUNK).astype(jnp.int32)
    out = _gather_rows(idx, table)
    return out.transpose(1, 0, 2)
